# baseline (speedup 1.0000x reference)
import functools

import jax
import jax.numpy as jnp
from jax import lax
from jax.experimental import pallas as pl
from jax.experimental.pallas import tpu as pltpu
from jax.experimental.pallas import tpu_sc as plsc

NUM_CLASSES = 100000
FEAT_DIM = 64
BATCH = 16384

_INFO = plsc.get_sparse_core_info()
_NC = _INFO.num_cores
_NS = _INFO.num_subcores
_NW = _NC * _NS
_BPW = BATCH // _NW
_IDX_ROWS = _BPW // 128

_sc_mesh = plsc.VectorSubcoreMesh(core_axis_name="c", subcore_axis_name="s")


@functools.partial(
    pl.kernel,
    mesh=_sc_mesh,
    out_type=jax.ShapeDtypeStruct((_NW, 16), jnp.float32),
    scratch_types=[
        pltpu.VMEM((_IDX_ROWS, 128), jnp.int32),
        pltpu.VMEM((_BPW, FEAT_DIM), jnp.float32),
        pltpu.VMEM((_BPW, FEAT_DIM), jnp.float32),
        pltpu.VMEM((16,), jnp.float32),
        pltpu.SemaphoreType.DMA,
    ],
    compiler_params=pltpu.CompilerParams(use_tc_tiling_on_sc=False),
)
def _sc_sw_partials(feat_hbm, y_hbm, centers_hbm, out_hbm,
                    idx_v, gat_v, feat_v, part_v, sem):
    wid = lax.axis_index("s") * _NC + lax.axis_index("c")
    base = wid * _BPW
    pltpu.sync_copy(y_hbm.at[pl.ds(wid * _IDX_ROWS, _IDX_ROWS)], idx_v)
    copies = []
    for k in range(_IDX_ROWS):
        copies.append(
            pltpu.async_copy(
                centers_hbm.at[idx_v.at[k]],
                gat_v.at[pl.ds(k * 128, 128)],
                sem,
            )
        )
    pltpu.sync_copy(feat_hbm.at[pl.ds(base, _BPW)], feat_v)
    for cp in copies:
        cp.wait()

    zero = jnp.zeros((16,), jnp.float32)

    def row_body(r, accs):
        a0, a1, a2, a3 = accs
        d0 = feat_v[r, pl.ds(0, 16)] - gat_v[r, pl.ds(0, 16)]
        d1 = feat_v[r, pl.ds(16, 16)] - gat_v[r, pl.ds(16, 16)]
        d2 = feat_v[r, pl.ds(32, 16)] - gat_v[r, pl.ds(32, 16)]
        d3 = feat_v[r, pl.ds(48, 16)] - gat_v[r, pl.ds(48, 16)]
        return (a0 + d0 * d0, a1 + d1 * d1, a2 + d2 * d2, a3 + d3 * d3)

    a0, a1, a2, a3 = lax.fori_loop(0, _BPW, row_body, (zero, zero, zero, zero))
    part_v[...] = (a0 + a1) + (a2 + a3)
    pltpu.sync_copy(part_v, out_hbm.at[wid])


_SB_BLK = 2000
_SB_GRID = NUM_CLASSES // _SB_BLK


def _tc_sb_body(centers_ref, mean_ref, sw_ref, out_ref, acc_ref):
    i = pl.program_id(0)

    @pl.when(i == 0)
    def _init():
        acc_ref[0] = 0.0

    d = centers_ref[...] - mean_ref[...]
    acc_ref[0] += jnp.sum(d * d)

    @pl.when(i == pl.num_programs(0) - 1)
    def _finish():
        sw = jnp.sum(sw_ref[...]) * (1.0 / BATCH)
        sb = (acc_ref[0] * 0.1) * 10.0
        out_ref[0, 0] = sw / sb


_tc_sb = pl.pallas_call(
    _tc_sb_body,
    grid=(_SB_GRID,),
    in_specs=[
        pl.BlockSpec((_SB_BLK, FEAT_DIM), lambda i: (i, 0)),
        pl.BlockSpec((1, FEAT_DIM), lambda i: (0, 0)),
        pl.BlockSpec((4, 128), lambda i: (0, 0)),
    ],
    out_specs=pl.BlockSpec(memory_space=pltpu.SMEM),
    out_shape=jax.ShapeDtypeStruct((1, 1), jnp.float32),
    scratch_shapes=[pltpu.SMEM((1,), jnp.float32)],
)


def kernel(feat, y, centers, feature_mean):
    y2 = y.astype(jnp.int32).reshape(_NW * _IDX_ROWS, 128)
    sw_parts = _sc_sw_partials(feat, y2, centers)
    out = _tc_sb(centers, feature_mean, sw_parts.reshape(4, 128))
    return out[0, 0]


# speedup vs baseline: 1.1417x; 1.1417x over previous
import functools

import jax
import jax.numpy as jnp
from jax import lax
from jax.experimental import pallas as pl
from jax.experimental.pallas import tpu as pltpu
from jax.experimental.pallas import tpu_sc as plsc

NUM_CLASSES = 100000
FEAT_DIM = 64
BATCH = 16384

_INFO = plsc.get_sparse_core_info()
_NC = _INFO.num_cores
_NS = _INFO.num_subcores
_NW = _NC * _NS
_IPW = BATCH // _NW
_GCH = 128
_NGC = _IPW // _GCH
_NGRP = _IPW // 16

_sc_mesh = plsc.VectorSubcoreMesh(core_axis_name="c", subcore_axis_name="s")


@functools.partial(
    pl.kernel,
    mesh=_sc_mesh,
    compiler_params=pltpu.CompilerParams(needs_layout_passes=False),
    out_type=jax.ShapeDtypeStruct((_NW, 16), jnp.float32),
    scratch_types=[
        pltpu.VMEM((_NGC, _GCH), jnp.int32),
        pltpu.VMEM((_IPW,), jnp.float32),
        pltpu.VMEM((2, _GCH, FEAT_DIM), jnp.float32),
        pltpu.VMEM((2, _GCH, 2 * FEAT_DIM), jnp.float32),
        pltpu.VMEM((16,), jnp.float32),
        pltpu.SemaphoreType.DMA,
        pltpu.SemaphoreType.DMA,
    ],
)
def _sc_sw(feat_hbm, gidx_hbm, par_hbm, pairs_hbm, out_sw,
           idx_v, par_v, f_v, g_v, sw_v, sem0, sem1):
    wid = lax.axis_index("s") * _NC + lax.axis_index("c")
    base = wid * _IPW
    sems = (sem0, sem1)

    pltpu.sync_copy(par_hbm.at[pl.ds(base, _IPW)], par_v)
    for ch in range(_NGC):
        pltpu.sync_copy(gidx_hbm.at[pl.ds(base + ch * _GCH, _GCH)],
                        idx_v.at[ch])

    def fire(ch):
        buf = ch % 2
        fd = pltpu.async_copy(
            feat_hbm.at[pl.ds(base + ch * _GCH, _GCH)], f_v.at[buf],
            sems[buf])
        gd = pltpu.async_copy(
            pairs_hbm.at[idx_v.at[ch]], g_v.at[buf], sems[buf])
        return (fd, gd)

    pend = [fire(0), fire(1)]

    lane = lax.iota(jnp.int32, 16)
    zero = jnp.zeros((16,), jnp.float32)

    def make_group(ch, buf):
        def group(g, accs):
            a0, a1, a2, a3 = accs
            pg = par_v[pl.ds(pl.multiple_of(ch * _GCH + g * 16, 16), 16)]
            for l in range(16):
                p = jnp.sum(jnp.where(lane == l, pg, 0.0))
                pb = jnp.full((16,), p, jnp.float32)
                r = g * 16 + l
                for k in range(4):
                    fe = f_v[buf, r, pl.ds(16 * k, 16)]
                    ce = g_v[buf, r, pl.ds(16 * k, 16)]
                    co = g_v[buf, r, pl.ds(64 + 16 * k, 16)]
                    d = (fe - ce) + pb * (ce - co)
                    if k == 0:
                        a0 = a0 + d * d
                    elif k == 1:
                        a1 = a1 + d * d
                    elif k == 2:
                        a2 = a2 + d * d
                    else:
                        a3 = a3 + d * d
            return (a0, a1, a2, a3)
        return group

    accs = (zero,) * 4
    for ch in range(_NGC):
        buf = ch % 2
        for dma in pend[buf]:
            dma.wait()
        accs = lax.fori_loop(0, _GCH // 16, make_group(ch, buf), accs)
        if ch + 2 < _NGC:
            pend[buf] = fire(ch + 2)

    a0, a1, a2, a3 = accs
    sw_v[...] = (a0 + a1) + (a2 + a3)
    pltpu.sync_copy(sw_v, out_sw.at[wid])


_SB_ROWS = 8192
_SB_GRID = (NUM_CLASSES + _SB_ROWS - 1) // _SB_ROWS


def _tc_sb_body(c_ref, m_ref, sb_ref):
    i = pl.program_id(0)

    @pl.when(i == 0)
    def _():
        sb_ref[0, 0] = 0.0

    blk = c_ref[...]
    row = lax.broadcasted_iota(jnp.int32, blk.shape, 0) + i * _SB_ROWS
    d = blk - m_ref[...]
    d = jnp.where(row < NUM_CLASSES, d, 0.0)
    sb_ref[0, 0] += jnp.sum(d * d)


_tc_sb = pl.pallas_call(
    _tc_sb_body,
    grid=(_SB_GRID,),
    in_specs=[
        pl.BlockSpec((_SB_ROWS, FEAT_DIM), lambda i: (i, 0)),
        pl.BlockSpec((1, FEAT_DIM), lambda i: (0, 0)),
    ],
    out_specs=pl.BlockSpec(memory_space=pltpu.SMEM),
    out_shape=jax.ShapeDtypeStruct((1, 1), jnp.float32),
)


def _combine_body(sw_ref, sb_ref, out_ref):
    sw = jnp.sum(sw_ref[...]) * (1.0 / BATCH)
    sb = (sb_ref[0, 0] * 0.1) * 10.0
    out_ref[0, 0] = jnp.exp(jnp.log(sw) - jnp.log(sb))


_combine = pl.pallas_call(
    _combine_body,
    in_specs=[
        pl.BlockSpec((_NW, 16), lambda: (0, 0)),
        pl.BlockSpec(memory_space=pltpu.SMEM),
    ],
    out_specs=pl.BlockSpec(memory_space=pltpu.SMEM),
    out_shape=jax.ShapeDtypeStruct((1, 1), jnp.float32),
)


def kernel(feat, y, centers, feature_mean):
    y32 = y.astype(jnp.int32)
    gidx = lax.shift_right_logical(y32, 1)
    par = (y32 & 1).astype(jnp.float32)
    pairs = centers.reshape(NUM_CLASSES // 2, 2 * FEAT_DIM)
    sw_p = _sc_sw(feat, gidx, par, pairs)
    sb = _tc_sb(centers, feature_mean)
    out = _combine(sw_p, sb)
    return out[0, 0]


# speedup vs baseline: 1.1641x; 1.0196x over previous
import functools

import jax
import jax.numpy as jnp
from jax import lax
from jax.experimental import pallas as pl
from jax.experimental.pallas import tpu as pltpu
from jax.experimental.pallas import tpu_sc as plsc

NUM_CLASSES = 100000
FEAT_DIM = 64
BATCH = 16384

_INFO = plsc.get_sparse_core_info()
_NC = _INFO.num_cores
_NS = _INFO.num_subcores
_NW = _NC * _NS
_IPW = BATCH // _NW
_GCH = 128
_NGC = _IPW // _GCH

_sc_mesh = plsc.VectorSubcoreMesh(core_axis_name="c", subcore_axis_name="s")


@functools.partial(
    pl.kernel,
    mesh=_sc_mesh,
    compiler_params=pltpu.CompilerParams(
        needs_layout_passes=False, use_tc_tiling_on_sc=False),
    out_type=jax.ShapeDtypeStruct((_NW, 16), jnp.float32),
    scratch_types=[
        pltpu.VMEM((_NGC, _GCH), jnp.int32),
        pltpu.VMEM((2, _GCH, FEAT_DIM), jnp.float32),
        pltpu.VMEM((2, _GCH, FEAT_DIM), jnp.float32),
        pltpu.VMEM((16,), jnp.float32),
        pltpu.SemaphoreType.DMA,
        pltpu.SemaphoreType.DMA,
    ],
)
def _sc_sw(feat_hbm, y_hbm, centers_hbm, out_sw,
           idx_v, f_v, g_v, sw_v, sem0, sem1):
    wid = lax.axis_index("s") * _NC + lax.axis_index("c")
    base = wid * _IPW
    sems = (sem0, sem1)

    for ch in range(_NGC):
        pltpu.sync_copy(y_hbm.at[pl.ds(base + ch * _GCH, _GCH)],
                        idx_v.at[ch])

    def fire(ch):
        buf = ch % 2
        fd = pltpu.async_copy(
            feat_hbm.at[pl.ds(base + ch * _GCH, _GCH)], f_v.at[buf],
            sems[buf])
        gd = pltpu.async_copy(
            centers_hbm.at[idx_v.at[ch]], g_v.at[buf], sems[buf])
        return (fd, gd)

    pend = [fire(0), fire(1)]

    zero = jnp.zeros((16,), jnp.float32)

    def make_group(buf):
        def group(g, accs):
            a0, a1, a2, a3 = accs
            for l in range(16):
                r = g * 16 + l
                for k in range(4):
                    d = (f_v[buf, r, pl.ds(16 * k, 16)]
                         - g_v[buf, r, pl.ds(16 * k, 16)])
                    if k == 0:
                        a0 = a0 + d * d
                    elif k == 1:
                        a1 = a1 + d * d
                    elif k == 2:
                        a2 = a2 + d * d
                    else:
                        a3 = a3 + d * d
            return (a0, a1, a2, a3)
        return group

    accs = (zero,) * 4
    for ch in range(_NGC):
        buf = ch % 2
        for dma in pend[buf]:
            dma.wait()
        accs = lax.fori_loop(0, _GCH // 16, make_group(buf), accs)
        if ch + 2 < _NGC:
            pend[buf] = fire(ch + 2)

    a0, a1, a2, a3 = accs
    sw_v[...] = (a0 + a1) + (a2 + a3)
    pltpu.sync_copy(sw_v, out_sw.at[wid])


_SB_ROWS = 8192
_SB_GRID = (NUM_CLASSES + _SB_ROWS - 1) // _SB_ROWS


def _tc_sb_body(c_ref, m_ref, sb_ref):
    i = pl.program_id(0)

    @pl.when(i == 0)
    def _():
        sb_ref[0, 0] = 0.0

    blk = c_ref[...]
    row = lax.broadcasted_iota(jnp.int32, blk.shape, 0) + i * _SB_ROWS
    d = blk - m_ref[...]
    d = jnp.where(row < NUM_CLASSES, d, 0.0)
    sb_ref[0, 0] += jnp.sum(d * d)


_tc_sb = pl.pallas_call(
    _tc_sb_body,
    grid=(_SB_GRID,),
    in_specs=[
        pl.BlockSpec((_SB_ROWS, FEAT_DIM), lambda i: (i, 0)),
        pl.BlockSpec((1, FEAT_DIM), lambda i: (0, 0)),
    ],
    out_specs=pl.BlockSpec(memory_space=pltpu.SMEM),
    out_shape=jax.ShapeDtypeStruct((1, 1), jnp.float32),
)


def _combine_body(sw_ref, sb_ref, out_ref):
    sw = jnp.sum(sw_ref[...]) * (1.0 / BATCH)
    sb = (sb_ref[0, 0] * 0.1) * 10.0
    out_ref[0, 0] = jnp.exp(jnp.log(sw) - jnp.log(sb))


_combine = pl.pallas_call(
    _combine_body,
    in_specs=[
        pl.BlockSpec((_NW, 16), lambda: (0, 0)),
        pl.BlockSpec(memory_space=pltpu.SMEM),
    ],
    out_specs=pl.BlockSpec(memory_space=pltpu.SMEM),
    out_shape=jax.ShapeDtypeStruct((1, 1), jnp.float32),
)


def kernel(feat, y, centers, feature_mean):
    sw_p = _sc_sw(feat, y.astype(jnp.int32), centers)
    sb = _tc_sb(centers, feature_mean)
    out = _combine(sw_p, sb)
    return out[0, 0]
